# Initial kernel scaffold; baseline (speedup 1.0000x reference)
import jax
import jax.numpy as jnp
from jax.experimental import pallas as pl


def kernel(x, w):
    raise NotImplementedError("write your pallas kernel here")


# speedup vs baseline: 5.8375x; 5.8375x over previous
import functools

import jax
import jax.numpy as jnp
from jax import lax
from jax.experimental import pallas as pl
from jax.experimental.pallas import tpu as pltpu
from jax.experimental.pallas import tpu_sc as plsc

_DIM = 64
_B = 1024 * 100
_NW = 32
_ROWS_PER_W = _B // _NW
_CHUNK = 128
_NCHUNK = _ROWS_PER_W // _CHUNK

_mesh = plsc.VectorSubcoreMesh(core_axis_name="c", subcore_axis_name="s")


@functools.partial(
    pl.kernel,
    mesh=_mesh,
    out_type=jax.ShapeDtypeStruct((_B, _DIM), jnp.float32),
    scratch_types=[
        pltpu.VMEM((_NCHUNK, _CHUNK), jnp.int32),
        pltpu.VMEM((_CHUNK, _DIM), jnp.float32),
        pltpu.SemaphoreType.DMA,
    ],
    compiler_params=pltpu.CompilerParams(use_tc_tiling_on_sc=False),
)
def _embed_sc(idx_hbm, table_hbm, out_hbm, idx_v, rows_v, gsem):
    wid = lax.axis_index("s") * 2 + lax.axis_index("c")
    pltpu.sync_copy(idx_hbm.at[wid], idx_v)
    base = wid * _ROWS_PER_W
    for i in range(_NCHUNK):
        pltpu.async_copy(table_hbm.at[idx_v.at[i]], rows_v, gsem).wait()
        pltpu.sync_copy(rows_v, out_hbm.at[pl.ds(base + i * _CHUNK, _CHUNK)])


def kernel(x, w):
    idx = x.reshape(_NW, _NCHUNK, _CHUNK).astype(jnp.int32)
    out = _embed_sc(idx, w)
    return out.reshape(x.shape[0], x.shape[1], _DIM)


# speedup vs baseline: 6.1332x; 1.0507x over previous
import functools

import jax
import jax.numpy as jnp
from jax import lax
from jax.experimental import pallas as pl
from jax.experimental.pallas import tpu as pltpu
from jax.experimental.pallas import tpu_sc as plsc

_DIM = 64
_B = 1024 * 100
_NW = 32
_ROWS_PER_W = _B // _NW
_CHUNK = 128
_NCHUNK = _ROWS_PER_W // _CHUNK
_NBUF = 4

_mesh = plsc.VectorSubcoreMesh(core_axis_name="c", subcore_axis_name="s")


@functools.partial(
    pl.kernel,
    mesh=_mesh,
    out_type=jax.ShapeDtypeStruct((_B, _DIM), jnp.float32),
    scratch_types=[
        pltpu.VMEM((_NCHUNK, _CHUNK), jnp.int32),
        [pltpu.VMEM((_CHUNK, _DIM), jnp.float32) for _ in range(_NBUF)],
        [pltpu.SemaphoreType.DMA for _ in range(_NBUF)],
        [pltpu.SemaphoreType.DMA for _ in range(_NBUF)],
    ],
    compiler_params=pltpu.CompilerParams(use_tc_tiling_on_sc=False),
)
def _embed_sc(idx_hbm, table_hbm, out_hbm, idx_v, bufs, gsems, ssems):
    wid = lax.axis_index("s") * 2 + lax.axis_index("c")
    pltpu.sync_copy(idx_hbm.at[wid], idx_v)
    base = wid * _ROWS_PER_W

    gathers = [None] * _NBUF
    stores = [None] * _NBUF

    def store(j):
        b = j % _NBUF
        gathers[b].wait()
        stores[b] = pltpu.async_copy(
            bufs[b], out_hbm.at[pl.ds(base + j * _CHUNK, _CHUNK)], ssems[b])

    for i in range(_NCHUNK):
        b = i % _NBUF
        if stores[b] is not None:
            stores[b].wait()
        gathers[b] = pltpu.async_copy(table_hbm.at[idx_v.at[i]], bufs[b], gsems[b])
        if i >= _NBUF - 1:
            store(i - (_NBUF - 1))
    for j in range(max(0, _NCHUNK - (_NBUF - 1)), _NCHUNK):
        store(j)
    for b in range(_NBUF):
        if stores[b] is not None:
            stores[b].wait()


def kernel(x, w):
    idx = x.reshape(_NW, _NCHUNK, _CHUNK).astype(jnp.int32)
    out = _embed_sc(idx, w)
    return out.reshape(x.shape[0], x.shape[1], _DIM)


# speedup vs baseline: 6.1476x; 1.0024x over previous
import functools

import jax
import jax.numpy as jnp
from jax import lax
from jax.experimental import pallas as pl
from jax.experimental.pallas import tpu as pltpu
from jax.experimental.pallas import tpu_sc as plsc

_DIM = 64
_BATCH = 1024
_SEQ = 100
_NW = 32
_B_PER_W = _BATCH // _NW
_NBUF = 4

_mesh = plsc.VectorSubcoreMesh(core_axis_name="c", subcore_axis_name="s")


@functools.partial(
    pl.kernel,
    mesh=_mesh,
    out_type=jax.ShapeDtypeStruct((_BATCH, _SEQ, _DIM), jnp.float32),
    scratch_types=[
        pltpu.VMEM((_B_PER_W, _SEQ), jnp.int32),
        [pltpu.VMEM((_SEQ, _DIM), jnp.float32) for _ in range(_NBUF)],
        [pltpu.SemaphoreType.DMA for _ in range(_NBUF)],
        [pltpu.SemaphoreType.DMA for _ in range(_NBUF)],
    ],
    compiler_params=pltpu.CompilerParams(use_tc_tiling_on_sc=False),
)
def _embed_sc(idx_hbm, table_hbm, out_hbm, idx_v, bufs, gsems, ssems):
    wid = lax.axis_index("s") * 2 + lax.axis_index("c")
    pltpu.sync_copy(idx_hbm.at[wid], idx_v)
    base = wid * _B_PER_W

    gathers = [None] * _NBUF
    stores = [None] * _NBUF

    def store(j):
        b = j % _NBUF
        gathers[b].wait()
        stores[b] = pltpu.async_copy(bufs[b], out_hbm.at[base + j], ssems[b])

    for i in range(_B_PER_W):
        b = i % _NBUF
        if stores[b] is not None:
            stores[b].wait()
        gathers[b] = pltpu.async_copy(table_hbm.at[idx_v.at[i]], bufs[b], gsems[b])
        if i >= _NBUF - 1:
            store(i - (_NBUF - 1))
    for j in range(max(0, _B_PER_W - (_NBUF - 1)), _B_PER_W):
        store(j)
    for b in range(_NBUF):
        if stores[b] is not None:
            stores[b].wait()


def kernel(x, w):
    idx = x.reshape(_NW, _B_PER_W, _SEQ).astype(jnp.int32)
    return _embed_sc(idx, w)


# speedup vs baseline: 9.3809x; 1.5259x over previous
import functools

import jax
import jax.numpy as jnp
from jax import lax
from jax.experimental import pallas as pl
from jax.experimental.pallas import tpu as pltpu
from jax.experimental.pallas import tpu_sc as plsc

_DIM = 64
_PAD = 128
_VOCAB = 1000
_BATCH = 1024
_SEQ = 100
_NW = 32
_B_PER_W = _BATCH // _NW
_NBUF = 3
_NPACK = 2

_mesh = plsc.VectorSubcoreMesh(core_axis_name="c", subcore_axis_name="s")


@functools.partial(
    pl.kernel,
    mesh=_mesh,
    out_type=jax.ShapeDtypeStruct((_BATCH, _SEQ, _DIM), jnp.float32),
    scratch_types=[
        pltpu.VMEM_SHARED((_VOCAB, _PAD), jnp.float32),
        pltpu.VMEM((_B_PER_W, _SEQ), jnp.int32),
        [pltpu.VMEM((_SEQ, _PAD), jnp.float32) for _ in range(_NBUF)],
        [pltpu.VMEM((_SEQ, _DIM), jnp.float32) for _ in range(_NPACK)],
        [pltpu.SemaphoreType.DMA for _ in range(_NBUF)],
        [pltpu.SemaphoreType.DMA for _ in range(_NPACK)],
    ],
)
def _embed_sc(idx_hbm, table_hbm, out_hbm, spt, idx_v, bufs, packs,
              gsems, ssems):
    cid = lax.axis_index("c")
    sid = lax.axis_index("s")
    wid = sid * 2 + cid

    @pl.when(sid == 0)
    def _():
        pltpu.sync_copy(table_hbm, spt)

    pltpu.sync_copy(idx_hbm.at[wid], idx_v)
    plsc.subcore_barrier()

    base = wid * _B_PER_W
    gathers = [None] * _NBUF
    stores = [None] * _NPACK

    def compact(big, small):
        def body(r, carry):
            for c in range(_DIM // 16):
                small[r, pl.ds(c * 16, 16)] = big[r, pl.ds(c * 16, 16)]
            return carry
        lax.fori_loop(0, _SEQ, body, 0)

    for b in range(min(_NBUF, _B_PER_W)):
        gathers[b] = pltpu.async_copy(spt.at[idx_v.at[b]], bufs[b], gsems[b])
    for i in range(_B_PER_W):
        b = i % _NBUF
        p = i % _NPACK
        gathers[b].wait()
        if stores[p] is not None:
            stores[p].wait()
        compact(bufs[b], packs[p])
        if i + _NBUF < _B_PER_W:
            gathers[b] = pltpu.async_copy(
                spt.at[idx_v.at[i + _NBUF]], bufs[b], gsems[b])
        stores[p] = pltpu.async_copy(packs[p], out_hbm.at[base + i], ssems[p])
    for p in range(_NPACK):
        if stores[p] is not None:
            stores[p].wait()


def kernel(x, w):
    idx = x.reshape(_NW, _B_PER_W, _SEQ).astype(jnp.int32)
    wp = jnp.pad(w, ((0, 0), (0, _PAD - _DIM)))
    return _embed_sc(idx, wp)
